# baseline (speedup 1.0000x reference)
import functools

import jax
import jax.numpy as jnp
from jax import lax
from jax.experimental import pallas as pl
from jax.experimental.pallas import tpu as pltpu
from jax.experimental.pallas import tpu_sc as plsc

VOCAB1 = 1000001
EMBED_DIM = 64
BATCH = 4096
SEQ = 200

NC = 2
NS = 16
NW = NC * NS

ROWS_PER_W = BATCH // NW
CHUNK = SEQ
N_CHUNKS = ROWS_PER_W
R = 4
LA = 2
N_ITERS = N_CHUNKS // R


def _build_kernel():
    mesh = plsc.VectorSubcoreMesh(core_axis_name="c", subcore_axis_name="s")

    @functools.partial(
        pl.kernel,
        mesh=mesh,
        out_type=jax.ShapeDtypeStruct((BATCH, SEQ, EMBED_DIM), jnp.float32),
        compiler_params=pltpu.CompilerParams(use_tc_tiling_on_sc=False),
        scratch_types=[
            pltpu.VMEM((N_CHUNKS, CHUNK), jnp.int32),
            pltpu.VMEM((R, CHUNK, EMBED_DIM), jnp.float32),
        ]
        + [pltpu.SemaphoreType.DMA] * (2 * R),
    )
    def k(table_hbm, ids_hbm, out_hbm, idx_v, rows_v, *sems):
        gsems = sems[:R]
        ssems = sems[R:]
        wid = lax.axis_index("s") * NC + lax.axis_index("c")
        base = wid * ROWS_PER_W
        pltpu.sync_copy(ids_hbm.at[pl.ds(base, ROWS_PER_W)], idx_v)

        def fire_gather(g, s):
            pltpu.async_copy(table_hbm.at[idx_v.at[g]], rows_v.at[s], gsems[s])

        def wait_gather(g, s):
            pltpu.make_async_copy(
                table_hbm.at[idx_v.at[g]], rows_v.at[s], gsems[s]
            ).wait()

        def fire_store(g, s):
            pltpu.async_copy(rows_v.at[s], out_hbm.at[base + g], ssems[s])

        def drain_store(s):
            pltpu.make_async_copy(rows_v.at[s], out_hbm.at[base], ssems[s]).wait()

        for g0 in range(LA):
            fire_gather(g0, g0)
        for p in range(R):
            sf = (p + LA) % R
            if p >= R - LA:
                drain_store(sf)
            fire_gather(p + LA, sf)
            wait_gather(p, p)
            fire_store(p, p)

        def body(i, carry):
            for p in range(R):
                g = i * R + p
                sf = (p + LA) % R
                drain_store(sf)
                fire_gather(g + LA, sf)
                wait_gather(g, p)
                fire_store(g, p)
            return carry

        lax.fori_loop(1, N_ITERS - 1, body, 0)

        gbase = (N_ITERS - 1) * R
        for p in range(R):
            g = gbase + p
            sf = (p + LA) % R
            drain_store(sf)
            if p < R - LA:
                fire_gather(g + LA, sf)
            wait_gather(g, p)
            fire_store(g, p)
        for p in range(R - LA, R):
            drain_store(p)

    return k


_k = _build_kernel()


@jax.jit
def kernel(input_ids, table):
    return _k(table, input_ids)
